# baseline (speedup 1.0000x reference)
import jax
import jax.numpy as jnp
from jax.experimental import pallas as pl
from jax.experimental.pallas import tpu as pltpu

_NEG = -9000000000000000.0
_SPLIT = 2


def _gat_step(x_ref, pos_ref, adj_ref, p_ref, out_ref, wh_ref, f2_ref):
    i = pl.program_id(1)
    bi = out_ref.shape[1]
    w = p_ref[0:128, :]
    a1r = p_ref[128:129, :]
    a2r = p_ref[129:130, :]
    wpt = p_ref[130:133, :]
    bp = p_ref[133:134, :]

    @pl.when(i == 0)
    def _per_batch():
        wh_all = jnp.dot(x_ref[0], w, preferred_element_type=jnp.float32)
        wh_ref[...] = wh_all
        f2_ref[...] = jax.lax.dot_general(
            a2r, wh_all, (((1,), (1,)), ((), ())),
            preferred_element_type=jnp.float32)

    wh = wh_ref[...]
    wh_i = wh_ref[pl.ds(i * bi, bi), :]
    f1 = jax.lax.dot_general(
        wh_i, a1r, (((1,), (1,)), ((), ())), preferred_element_type=jnp.float32)
    e = f1 + f2_ref[...]
    e = jnp.maximum(e, 0.2 * e)
    e = jnp.where(adj_ref[0] > 0.0, e, _NEG)
    m = jnp.max(e, axis=1, keepdims=True)
    p = jnp.exp(e - m)
    att = p * (1.0 / jnp.sum(p, axis=1, keepdims=True))
    h = jnp.dot(att, wh, preferred_element_type=jnp.float32)
    pe = jnp.dot(pos_ref[0], wpt, preferred_element_type=jnp.float32)
    pe = jnp.maximum(pe + bp, 0.0)
    h = h + pe
    out_ref[0] = jnp.where(h > 0.0, h, jnp.exp(h) - 1.0)


def kernel(x, pos, adj, W, a, W_pos, b_pos):
    B, N, F_in = x.shape
    F_out = W.shape[1]
    packed = jnp.concatenate(
        [W,
         a[:F_out, 0][None, :],
         a[F_out:, 0][None, :],
         W_pos.T,
         b_pos[None, :]],
        axis=0)

    bi = N // _SPLIT
    return pl.pallas_call(
        _gat_step,
        grid=(B, _SPLIT),
        in_specs=[
            pl.BlockSpec((1, N, F_in), lambda b, i: (b, 0, 0)),
            pl.BlockSpec((1, bi, 3), lambda b, i: (b, i, 0)),
            pl.BlockSpec((1, bi, N), lambda b, i: (b, i, 0)),
            pl.BlockSpec((F_in + 6, F_out), lambda b, i: (0, 0)),
        ],
        out_specs=pl.BlockSpec((1, bi, F_out), lambda b, i: (b, i, 0)),
        out_shape=jax.ShapeDtypeStruct((B, N, F_out), jnp.float32),
        scratch_shapes=[
            pltpu.VMEM((N, F_out), jnp.float32),
            pltpu.VMEM((1, N), jnp.float32),
        ],
    )(x, pos, adj, packed)


# speedup vs baseline: 1.0202x; 1.0202x over previous
import jax
import jax.numpy as jnp
from jax.experimental import pallas as pl

_NEG = -9000000000000000.0
_STRIPS = 8


def _gat_step(x_ref, pos_ref, *rest):
    adj_refs = rest[:_STRIPS]
    p_ref = rest[_STRIPS]
    out_ref = rest[_STRIPS + 1]
    w = p_ref[0:128, :]
    a1r = p_ref[128:129, :]
    a2r = p_ref[129:130, :]
    wpt = p_ref[130:133, :]
    bp = p_ref[133:134, :]
    wh = jnp.dot(x_ref[0], w, preferred_element_type=jnp.float32)
    f1 = jax.lax.dot_general(
        wh, a1r, (((1,), (1,)), ((), ())), preferred_element_type=jnp.float32)
    f2 = jax.lax.dot_general(
        a2r, wh, (((1,), (1,)), ((), ())), preferred_element_type=jnp.float32)
    pe = jnp.dot(pos_ref[0], wpt, preferred_element_type=jnp.float32)
    pe = jnp.maximum(pe + bp, 0.0)

    n = wh.shape[0]
    s = n // _STRIPS
    for k, adj_ref in enumerate(adj_refs):
        e = f1[k * s:(k + 1) * s] + f2
        e = jnp.maximum(e, 0.2 * e)
        e = jnp.where(adj_ref[0] > 0.0, e, _NEG)
        m = jnp.max(e, axis=1, keepdims=True)
        p = jnp.exp(e - m)
        att = p * (1.0 / jnp.sum(p, axis=1, keepdims=True))
        h = jnp.dot(att, wh, preferred_element_type=jnp.float32)
        h = h + pe[k * s:(k + 1) * s]
        out_ref[0, k * s:(k + 1) * s] = jnp.where(h > 0.0, h, jnp.exp(h) - 1.0)


def kernel(x, pos, adj, W, a, W_pos, b_pos):
    B, N, F_in = x.shape
    F_out = W.shape[1]
    packed = jnp.concatenate(
        [W,
         a[:F_out, 0][None, :],
         a[F_out:, 0][None, :],
         W_pos.T,
         b_pos[None, :]],
        axis=0)

    s = N // _STRIPS
    adj_specs = [
        pl.BlockSpec((1, s, N), lambda b, _k=k: (b, _k, 0))
        for k in range(_STRIPS)
    ]
    return pl.pallas_call(
        _gat_step,
        grid=(B,),
        in_specs=[
            pl.BlockSpec((1, N, F_in), lambda b: (b, 0, 0)),
            pl.BlockSpec((1, N, 3), lambda b: (b, 0, 0)),
            *adj_specs,
            pl.BlockSpec((F_in + 6, F_out), lambda b: (0, 0)),
        ],
        out_specs=pl.BlockSpec((1, N, F_out), lambda b: (b, 0, 0)),
        out_shape=jax.ShapeDtypeStruct((B, N, F_out), jnp.float32),
    )(x, pos, *([adj] * _STRIPS), packed)
